# baseline (speedup 1.0000x reference)
import jax
import jax.numpy as jnp
from jax.experimental import pallas as pl

_EMB = 3072
_N = 1024
_M_BLK = 512


def _proj_kernel(x_ref, v_ref, bb_ref, kp_ref, we_ref, wv_ref, wb_ref, wk_ref,
                 b_ref, m_ref, o_ref):
    dims = (((1,), (0,)), ((), ()))
    acc = jax.lax.dot_general(
        x_ref[...].astype(jnp.bfloat16), we_ref[...], dims,
        preferred_element_type=jnp.float32)
    acc += jax.lax.dot_general(
        v_ref[...].astype(jnp.bfloat16), wv_ref[...], dims,
        preferred_element_type=jnp.float32)
    acc += jax.lax.dot_general(
        bb_ref[...].astype(jnp.bfloat16), wb_ref[...], dims,
        preferred_element_type=jnp.float32)
    acc += jax.lax.dot_general(
        kp_ref[...].astype(jnp.bfloat16), wk_ref[...], dims,
        preferred_element_type=jnp.float32)
    o_ref[...] = (acc + b_ref[...]) * m_ref[...]


def kernel(embeddings, visibility_scores, bbox_ltwh, keypoints_xyc, feats_masks, W, b):
    bsz, slen = feats_masks.shape
    m_rows = bsz * slen

    x = embeddings.reshape(m_rows, _EMB)
    vis = visibility_scores.reshape(m_rows, 6)
    bb = bbox_ltwh.reshape(m_rows, 4)
    kp = keypoints_xyc.reshape(m_rows, 51)
    mask = feats_masks.reshape(m_rows, 1).astype(jnp.float32)
    bias = b.reshape(1, _N)

    wt = W.T.astype(jnp.bfloat16)
    w_emb = wt[:_EMB]
    w_vis = wt[_EMB:_EMB + 6]
    w_bb = wt[_EMB + 6:_EMB + 10]
    w_kp = wt[_EMB + 10:]

    grid = (m_rows // _M_BLK,)
    full = lambda a: pl.BlockSpec(a.shape, lambda m: (0,) * a.ndim)
    out = pl.pallas_call(
        _proj_kernel,
        grid=grid,
        in_specs=[
            pl.BlockSpec((_M_BLK, _EMB), lambda m: (m, 0)),
            pl.BlockSpec((_M_BLK, 6), lambda m: (m, 0)),
            pl.BlockSpec((_M_BLK, 4), lambda m: (m, 0)),
            pl.BlockSpec((_M_BLK, 51), lambda m: (m, 0)),
            full(w_emb), full(w_vis), full(w_bb), full(w_kp),
            full(bias),
            pl.BlockSpec((_M_BLK, 1), lambda m: (m, 0)),
        ],
        out_specs=pl.BlockSpec((_M_BLK, _N), lambda m: (m, 0)),
        out_shape=jax.ShapeDtypeStruct((m_rows, _N), jnp.float32),
    )(x, vis, bb, kp, w_emb, w_vis, w_bb, w_kp, bias, mask)

    return out.reshape(bsz, slen, _N)


# speedup vs baseline: 1.0721x; 1.0721x over previous
import jax
import jax.numpy as jnp
from jax.experimental import pallas as pl

_EMB = 3072
_SMALL = 61
_K_PAD = 3200
_N = 1024
_M_BLK = 512


def _proj_kernel(x_ref, v_ref, bb_ref, kp_ref, w_ref, b_ref, m_ref, o_ref):
    i = pl.program_id(0)
    r = pl.ds(i * _M_BLK, _M_BLK)
    x16 = x_ref[...].astype(jnp.bfloat16)
    sm = jnp.concatenate(
        [v_ref[r, :].astype(jnp.bfloat16),
         bb_ref[r, :].astype(jnp.bfloat16),
         kp_ref[r, :].astype(jnp.bfloat16),
         jnp.zeros((_M_BLK, _K_PAD - _EMB - _SMALL), jnp.bfloat16)],
        axis=1)
    xa = jnp.concatenate([x16, sm], axis=1)
    acc = jax.lax.dot_general(
        xa, w_ref[...], (((1,), (0,)), ((), ())),
        preferred_element_type=jnp.float32)
    o_ref[...] = (acc + b_ref[...]) * m_ref[...]


def kernel(embeddings, visibility_scores, bbox_ltwh, keypoints_xyc, feats_masks, W, b):
    bsz, slen = feats_masks.shape
    m_rows = bsz * slen

    x = embeddings.reshape(m_rows, _EMB)
    vis = visibility_scores.reshape(m_rows, 6)
    bb = bbox_ltwh.reshape(m_rows, 4)
    kp = keypoints_xyc.reshape(m_rows, 51)
    mask = feats_masks.reshape(m_rows, 1).astype(jnp.float32)
    bias = b.reshape(1, _N)

    wt = W.T.astype(jnp.bfloat16)
    w_full = jnp.concatenate(
        [wt, jnp.zeros((_K_PAD - _EMB - _SMALL, _N), jnp.bfloat16)], axis=0)

    grid = (m_rows // _M_BLK,)
    out = pl.pallas_call(
        _proj_kernel,
        grid=grid,
        in_specs=[
            pl.BlockSpec((_M_BLK, _EMB), lambda m: (m, 0)),
            pl.BlockSpec((m_rows, 6), lambda m: (0, 0)),
            pl.BlockSpec((m_rows, 4), lambda m: (0, 0)),
            pl.BlockSpec((m_rows, 51), lambda m: (0, 0)),
            pl.BlockSpec((_K_PAD, _N), lambda m: (0, 0)),
            pl.BlockSpec((1, _N), lambda m: (0, 0)),
            pl.BlockSpec((_M_BLK, 1), lambda m: (m, 0)),
        ],
        out_specs=pl.BlockSpec((_M_BLK, _N), lambda m: (m, 0)),
        out_shape=jax.ShapeDtypeStruct((m_rows, _N), jnp.float32),
    )(x, vis, bb, kp, w_full, bias, mask)

    return out.reshape(bsz, slen, _N)


# speedup vs baseline: 1.0893x; 1.0161x over previous
import jax
import jax.numpy as jnp
from jax.experimental import pallas as pl

_EMB = 3072
_SMALL = 61
_SMALL_PAD = 128
_N = 1024
_M_BLK = 512


def _proj_kernel(x_ref, v_ref, bb_ref, kp_ref, we_ref, ws_ref, b_ref, m_ref, o_ref):
    i = pl.program_id(0)
    r = pl.ds(i * _M_BLK, _M_BLK)
    dims = (((1,), (0,)), ((), ()))
    acc = jax.lax.dot_general(
        x_ref[...].astype(jnp.bfloat16), we_ref[...], dims,
        preferred_element_type=jnp.float32)
    sm = jnp.concatenate(
        [v_ref[r, :].astype(jnp.bfloat16),
         bb_ref[r, :].astype(jnp.bfloat16),
         kp_ref[r, :].astype(jnp.bfloat16),
         jnp.zeros((_M_BLK, _SMALL_PAD - _SMALL), jnp.bfloat16)],
        axis=1)
    acc += jax.lax.dot_general(sm, ws_ref[...], dims,
                               preferred_element_type=jnp.float32)
    o_ref[...] = (acc + b_ref[...]) * m_ref[...]


def kernel(embeddings, visibility_scores, bbox_ltwh, keypoints_xyc, feats_masks, W, b):
    bsz, slen = feats_masks.shape
    m_rows = bsz * slen

    x = embeddings.reshape(m_rows, _EMB)
    vis = visibility_scores.reshape(m_rows, 6)
    bb = bbox_ltwh.reshape(m_rows, 4)
    kp = keypoints_xyc.reshape(m_rows, 51)
    mask = feats_masks.reshape(m_rows, 1).astype(jnp.float32)
    bias = b.reshape(1, _N)

    wt = W.T.astype(jnp.bfloat16)
    w_emb = wt[:_EMB]
    w_small = jnp.concatenate(
        [wt[_EMB:], jnp.zeros((_SMALL_PAD - _SMALL, _N), jnp.bfloat16)], axis=0)

    grid = (m_rows // _M_BLK,)
    out = pl.pallas_call(
        _proj_kernel,
        grid=grid,
        in_specs=[
            pl.BlockSpec((_M_BLK, _EMB), lambda m: (m, 0)),
            pl.BlockSpec((m_rows, 6), lambda m: (0, 0)),
            pl.BlockSpec((m_rows, 4), lambda m: (0, 0)),
            pl.BlockSpec((m_rows, 51), lambda m: (0, 0)),
            pl.BlockSpec((_EMB, _N), lambda m: (0, 0)),
            pl.BlockSpec((_SMALL_PAD, _N), lambda m: (0, 0)),
            pl.BlockSpec((1, _N), lambda m: (0, 0)),
            pl.BlockSpec((_M_BLK, 1), lambda m: (m, 0)),
        ],
        out_specs=pl.BlockSpec((_M_BLK, _N), lambda m: (m, 0)),
        out_shape=jax.ShapeDtypeStruct((m_rows, _N), jnp.float32),
    )(x, vis, bb, kp, w_emb, w_small, bias, mask)

    return out.reshape(bsz, slen, _N)
